# baseline (speedup 1.0000x reference)
import functools

import jax
import jax.numpy as jnp
from jax import lax
from jax.experimental import pallas as pl
from jax.experimental.pallas import tpu as pltpu
from jax.experimental.pallas import tpu_sc as plsc

VOCAB = 100000
D = 128
B = 4096
L = 200
ROWS = B * L
NC, NS, LANES = 2, 16, 16
NW = NC * NS
PER_W = ROWS // NW
CHUNK = 128
N_CH = PER_W // CHUNK

_R = 1000
_HI = lax.Precision.HIGHEST


def _proj_body(src_ref, wf_ref, w1_ref, fpw_ref, w2in_ref, fpb_ref, cpb_ref,
               p1_ref, p2_ref):
    w2 = jnp.dot(fpw_ref[...], w2in_ref[...], precision=_HI)
    bias = jnp.dot(fpb_ref[...], w2in_ref[...], precision=_HI) + cpb_ref[...]
    p1_ref[...] = jnp.dot(src_ref[...], w1_ref[...])
    p2_ref[...] = jnp.dot(wf_ref[...], w2) + bias


def _project_tables(src_table, wf_table, w1, fpw, w2in, fpb, cpb):
    grid = (VOCAB // _R,)
    return pl.pallas_call(
        _proj_body,
        grid=grid,
        in_specs=[
            pl.BlockSpec((_R, D), lambda i: (i, 0)),
            pl.BlockSpec((_R, D), lambda i: (i, 0)),
            pl.BlockSpec((D, D), lambda i: (0, 0)),
            pl.BlockSpec((D, 32), lambda i: (0, 0)),
            pl.BlockSpec((32, D), lambda i: (0, 0)),
            pl.BlockSpec((1, 32), lambda i: (0, 0)),
            pl.BlockSpec((1, D), lambda i: (0, 0)),
        ],
        out_specs=[
            pl.BlockSpec((_R, D), lambda i: (i, 0)),
            pl.BlockSpec((_R, D), lambda i: (i, 0)),
        ],
        out_shape=[
            jax.ShapeDtypeStruct((VOCAB, D), jnp.float32),
            jax.ShapeDtypeStruct((VOCAB, D), jnp.float32),
        ],
    )(src_table, wf_table, w1, fpw, w2in, fpb, cpb)


@functools.partial(
    pl.kernel,
    out_type=jax.ShapeDtypeStruct((ROWS, D), jnp.float32),
    mesh=plsc.VectorSubcoreMesh(core_axis_name="c", subcore_axis_name="s"),
    scratch_types=[
        pltpu.VMEM((N_CH, CHUNK), jnp.int32),
        pltpu.VMEM((N_CH, CHUNK), jnp.int32),
        pltpu.VMEM((CHUNK,), jnp.float32),
        pltpu.VMEM((CHUNK,), jnp.float32),
        pltpu.VMEM((CHUNK, D), jnp.float32),
        pltpu.VMEM((CHUNK, D), jnp.float32),
        pltpu.VMEM((CHUNK, D), jnp.float32),
        pltpu.VMEM((CHUNK, D), jnp.float32),
        pltpu.SemaphoreType.DMA,
        pltpu.SemaphoreType.DMA,
        pltpu.SemaphoreType.DMA,
        pltpu.SemaphoreType.DMA,
        pltpu.SemaphoreType.DMA,
        pltpu.SemaphoreType.DMA,
        pltpu.SemaphoreType.DMA,
        pltpu.SemaphoreType.DMA,
        pltpu.SemaphoreType.DMA,
    ],
)
def _sc_gather_add(tok_hbm, wf_hbm, notes_hbm, p1_hbm, p2_hbm, out_hbm,
                   idx_t_all, idx_w_all, notes0, notes1,
                   rows1_0, rows1_1, rows2_0, rows2_1,
                   isem, nsem0, nsem1, g1s0, g1s1, g2s0, g2s1, ws0, ws1):
    notes_v = (notes0, notes1)
    rows1 = (rows1_0, rows1_1)
    rows2 = (rows2_0, rows2_1)
    nsem = (nsem0, nsem1)
    g1s = (g1s0, g1s1)
    g2s = (g2s0, g2s1)
    ws = (ws0, ws1)

    wid = lax.axis_index("s") * NC + lax.axis_index("c")
    base_w = wid * PER_W
    crow = wid * N_CH
    lanes = lax.iota(jnp.int32, LANES)

    ct = pltpu.async_copy(tok_hbm.at[pl.ds(crow, N_CH), :], idx_t_all, isem)
    cw = pltpu.async_copy(wf_hbm.at[pl.ds(crow, N_CH), :], idx_w_all, isem)
    ct.wait()
    cw.wait()

    def fire(c, b):
        base = base_w + c * CHUNK
        pltpu.async_copy(notes_hbm.at[pl.ds(base, CHUNK)], notes_v[b], nsem[b])
        pltpu.async_copy(p1_hbm.at[idx_t_all.at[c]], rows1[b], g1s[b])
        pltpu.async_copy(p2_hbm.at[idx_w_all.at[c]], rows2[b], g2s[b])

    def wait_gathers(b):
        pltpu.make_async_copy(notes_hbm.at[pl.ds(0, CHUNK)], notes_v[b], nsem[b]).wait()
        pltpu.make_async_copy(p1_hbm.at[idx_t_all.at[0]], rows1[b], g1s[b]).wait()
        pltpu.make_async_copy(p2_hbm.at[idx_w_all.at[0]], rows2[b], g2s[b]).wait()

    def fire_out(c, b):
        base = base_w + c * CHUNK
        pltpu.async_copy(rows1[b], out_hbm.at[pl.ds(base, CHUNK), :], ws[b])

    def wait_out(b):
        pltpu.make_async_copy(rows1[b], out_hbm.at[pl.ds(0, CHUNK), :], ws[b]).wait()

    def compute(b):
        r1, r2, nv_ref = rows1[b], rows2[b], notes_v[b]

        @plsc.parallel_loop(0, CHUNK, unroll=4)
        def _(row):
            for g in range(D // LANES):
                sl = pl.ds(g * LANES, LANES)
                plsc.addupdate(r1.at[row, sl], r2[row, sl])

        @plsc.parallel_loop(0, CHUNK // LANES, unroll=2)
        def _(q):
            nv = nv_ref[pl.ds(q * LANES, LANES)]
            sl = pl.ds(D - LANES, LANES)
            for r in range(LANES):
                vec = jnp.where(lanes == LANES - 1,
                                jnp.broadcast_to(nv[r], (LANES,)), 0.0)
                plsc.addupdate(r1.at[q * LANES + r, sl], vec)

    fire(0, 0)

    def body(i, carry):
        for bslot in range(2):
            c = 2 * i + bslot
            q = 1 - bslot

            @pl.when(c >= 1)
            def _():
                wait_out(q)

            @pl.when(c + 1 < N_CH)
            def _():
                fire(c + 1, q)

            wait_gathers(bslot)
            compute(bslot)
            fire_out(c, bslot)
        return carry

    lax.fori_loop(0, N_CH // 2, body, 0)
    wait_out(1)


def kernel(token, weighted_factor, weighted_notes, src_table, wf_table,
           fp_W, fp_b, cp_W, cp_b):
    w1 = jnp.pad(cp_W[:D, :], ((0, 0), (0, 1)))
    w2in = jnp.pad(cp_W[D:, :], ((0, 7), (0, 1)))
    fpw = jnp.pad(fp_W, ((0, 0), (0, 7)))
    fpb = jnp.pad(fp_b, (0, 7)).reshape(1, 32)
    cpb = jnp.pad(cp_b, (0, 1)).reshape(1, D)

    p1, p2 = _project_tables(src_table, wf_table, w1, fpw, w2in, fpb, cpb)

    tok = token.reshape(NW * N_CH, CHUNK).astype(jnp.int32)
    wf = weighted_factor.reshape(NW * N_CH, CHUNK).astype(jnp.int32)
    notes = weighted_notes.reshape(ROWS)
    out = _sc_gather_add(tok, wf, notes, p1, p2)
    return out.reshape(B, L, D)


# speedup vs baseline: 1.1026x; 1.1026x over previous
import functools

import jax
import jax.numpy as jnp
from jax import lax
from jax.experimental import pallas as pl
from jax.experimental.pallas import tpu as pltpu
from jax.experimental.pallas import tpu_sc as plsc

VOCAB = 100000
D = 128
B = 4096
L = 200
ROWS = B * L
NC, NS, LANES = 2, 16, 16
NW = NC * NS
PER_W = ROWS // NW
CHUNK = 128
N_CH = PER_W // CHUNK

_R = 4000
_HI = lax.Precision.HIGHEST


def _proj_body(src_ref, wf_ref, w1_ref, fpw_ref, w2in_ref, fpb_ref, cpb_ref,
               p1_ref, p2_ref):
    w2 = jnp.dot(fpw_ref[...], w2in_ref[...], precision=_HI)
    bias = jnp.dot(fpb_ref[...], w2in_ref[...], precision=_HI) + cpb_ref[...]
    p1_ref[...] = jnp.dot(src_ref[...], w1_ref[...])
    p2_ref[...] = jnp.dot(wf_ref[...], w2) + bias


def _project_tables(src_table, wf_table, w1, fpw, w2in, fpb, cpb):
    grid = (VOCAB // _R,)
    return pl.pallas_call(
        _proj_body,
        grid=grid,
        in_specs=[
            pl.BlockSpec((_R, D), lambda i: (i, 0)),
            pl.BlockSpec((_R, D), lambda i: (i, 0)),
            pl.BlockSpec((D, D), lambda i: (0, 0)),
            pl.BlockSpec((D, 32), lambda i: (0, 0)),
            pl.BlockSpec((32, D), lambda i: (0, 0)),
            pl.BlockSpec((1, 32), lambda i: (0, 0)),
            pl.BlockSpec((1, D), lambda i: (0, 0)),
        ],
        out_specs=[
            pl.BlockSpec((_R, D), lambda i: (i, 0)),
            pl.BlockSpec((_R, D), lambda i: (i, 0)),
        ],
        out_shape=[
            jax.ShapeDtypeStruct((VOCAB, D), jnp.float32),
            jax.ShapeDtypeStruct((VOCAB, D), jnp.float32),
        ],
    )(src_table, wf_table, w1, fpw, w2in, fpb, cpb)


@functools.partial(
    pl.kernel,
    out_type=jax.ShapeDtypeStruct((ROWS, D), jnp.float32),
    mesh=plsc.VectorSubcoreMesh(core_axis_name="c", subcore_axis_name="s"),
    scratch_types=[
        pltpu.VMEM((N_CH, CHUNK), jnp.int32),
        pltpu.VMEM((N_CH, CHUNK), jnp.int32),
        pltpu.VMEM((CHUNK,), jnp.float32),
        pltpu.VMEM((CHUNK,), jnp.float32),
        pltpu.VMEM((CHUNK, D), jnp.float32),
        pltpu.VMEM((CHUNK, D), jnp.float32),
        pltpu.VMEM((CHUNK, D), jnp.float32),
        pltpu.VMEM((CHUNK, D), jnp.float32),
        pltpu.SemaphoreType.DMA,
        pltpu.SemaphoreType.DMA,
        pltpu.SemaphoreType.DMA,
        pltpu.SemaphoreType.DMA,
        pltpu.SemaphoreType.DMA,
        pltpu.SemaphoreType.DMA,
        pltpu.SemaphoreType.DMA,
        pltpu.SemaphoreType.DMA,
        pltpu.SemaphoreType.DMA,
    ],
)
def _sc_gather_add(tok_hbm, wf_hbm, notes_hbm, p1_hbm, p2_hbm, out_hbm,
                   idx_t_all, idx_w_all, notes0, notes1,
                   rows1_0, rows1_1, rows2_0, rows2_1,
                   isem, nsem0, nsem1, g1s0, g1s1, g2s0, g2s1, ws0, ws1):
    notes_v = (notes0, notes1)
    rows1 = (rows1_0, rows1_1)
    rows2 = (rows2_0, rows2_1)
    nsem = (nsem0, nsem1)
    g1s = (g1s0, g1s1)
    g2s = (g2s0, g2s1)
    ws = (ws0, ws1)

    wid = lax.axis_index("s") * NC + lax.axis_index("c")
    base_w = wid * PER_W
    crow = wid * N_CH
    lanes = lax.iota(jnp.int32, LANES)

    ct = pltpu.async_copy(tok_hbm.at[pl.ds(crow, N_CH), :], idx_t_all, isem)
    cw = pltpu.async_copy(wf_hbm.at[pl.ds(crow, N_CH), :], idx_w_all, isem)
    ct.wait()
    cw.wait()

    def fire(c, b):
        base = base_w + c * CHUNK
        pltpu.async_copy(notes_hbm.at[pl.ds(base, CHUNK)], notes_v[b], nsem[b])
        pltpu.async_copy(p1_hbm.at[idx_t_all.at[c]], rows1[b], g1s[b])
        pltpu.async_copy(p2_hbm.at[idx_w_all.at[c]], rows2[b], g2s[b])

    def wait_gathers(b):
        pltpu.make_async_copy(notes_hbm.at[pl.ds(0, CHUNK)], notes_v[b], nsem[b]).wait()
        pltpu.make_async_copy(p1_hbm.at[idx_t_all.at[0]], rows1[b], g1s[b]).wait()
        pltpu.make_async_copy(p2_hbm.at[idx_w_all.at[0]], rows2[b], g2s[b]).wait()

    def fire_out(c, b):
        base = base_w + c * CHUNK
        pltpu.async_copy(rows1[b], out_hbm.at[pl.ds(base, CHUNK), :], ws[b])

    def wait_out(b):
        pltpu.make_async_copy(rows1[b], out_hbm.at[pl.ds(0, CHUNK), :], ws[b]).wait()

    def compute(b):
        r1, r2, nv_ref = rows1[b], rows2[b], notes_v[b]

        @plsc.parallel_loop(0, CHUNK, unroll=4)
        def _(row):
            for g in range(D // LANES):
                sl = pl.ds(g * LANES, LANES)
                plsc.addupdate(r1.at[row, sl], r2[row, sl])

        @plsc.parallel_loop(0, CHUNK // LANES, unroll=2)
        def _(q):
            nv = nv_ref[pl.ds(q * LANES, LANES)]
            sl = pl.ds(D - LANES, LANES)
            for r in range(LANES):
                vec = jnp.where(lanes == LANES - 1,
                                jnp.broadcast_to(nv[r], (LANES,)), 0.0)
                plsc.addupdate(r1.at[q * LANES + r, sl], vec)

    fire(0, 0)

    def body(i, carry):
        for bslot in range(2):
            c = 2 * i + bslot
            q = 1 - bslot

            @pl.when(c >= 1)
            def _():
                wait_out(q)

            @pl.when(c + 1 < N_CH)
            def _():
                fire(c + 1, q)

            wait_gathers(bslot)
            compute(bslot)
            fire_out(c, bslot)
        return carry

    lax.fori_loop(0, N_CH // 2, body, 0)
    wait_out(1)


def kernel(token, weighted_factor, weighted_notes, src_table, wf_table,
           fp_W, fp_b, cp_W, cp_b):
    w1 = jnp.pad(cp_W[:D, :], ((0, 0), (0, 1)))
    w2in = jnp.pad(cp_W[D:, :], ((0, 7), (0, 1)))
    fpw = jnp.pad(fp_W, ((0, 0), (0, 7)))
    fpb = jnp.pad(fp_b, (0, 7)).reshape(1, 32)
    cpb = jnp.pad(cp_b, (0, 1)).reshape(1, D)

    p1, p2 = _project_tables(src_table, wf_table, w1, fpw, w2in, fpb, cpb)

    tok = token.reshape(NW * N_CH, CHUNK).astype(jnp.int32)
    wf = weighted_factor.reshape(NW * N_CH, CHUNK).astype(jnp.int32)
    notes = weighted_notes.reshape(ROWS)
    out = _sc_gather_add(tok, wf, notes, p1, p2)
    return out.reshape(B, L, D)


# speedup vs baseline: 1.1074x; 1.0044x over previous
import functools

import jax
import jax.numpy as jnp
from jax import lax
from jax.experimental import pallas as pl
from jax.experimental.pallas import tpu as pltpu
from jax.experimental.pallas import tpu_sc as plsc

VOCAB = 100000
D = 128
B = 4096
L = 200
ROWS = B * L
NC, NS, LANES = 2, 16, 16
NW = NC * NS
PER_W = ROWS // NW
CHUNK = 128
N_CH = PER_W // CHUNK

_R = 10000
_HI = lax.Precision.HIGHEST


def _proj_body(src_ref, wf_ref, w1_ref, fpw_ref, w2in_ref, fpb_ref, cpb_ref,
               p1_ref, p2_ref):
    w2 = jnp.dot(fpw_ref[...], w2in_ref[...], precision=_HI)
    bias = jnp.dot(fpb_ref[...], w2in_ref[...], precision=_HI) + cpb_ref[...]
    p1_ref[...] = jnp.dot(src_ref[...], w1_ref[...])
    p2_ref[...] = jnp.dot(wf_ref[...], w2) + bias


def _project_tables(src_table, wf_table, w1, fpw, w2in, fpb, cpb):
    grid = (VOCAB // _R,)
    return pl.pallas_call(
        _proj_body,
        grid=grid,
        in_specs=[
            pl.BlockSpec((_R, D), lambda i: (i, 0)),
            pl.BlockSpec((_R, D), lambda i: (i, 0)),
            pl.BlockSpec((D, D), lambda i: (0, 0)),
            pl.BlockSpec((D, 32), lambda i: (0, 0)),
            pl.BlockSpec((32, D), lambda i: (0, 0)),
            pl.BlockSpec((1, 32), lambda i: (0, 0)),
            pl.BlockSpec((1, D), lambda i: (0, 0)),
        ],
        out_specs=[
            pl.BlockSpec((_R, D), lambda i: (i, 0)),
            pl.BlockSpec((_R, D), lambda i: (i, 0)),
        ],
        out_shape=[
            jax.ShapeDtypeStruct((VOCAB, D), jnp.float32),
            jax.ShapeDtypeStruct((VOCAB, D), jnp.float32),
        ],
    )(src_table, wf_table, w1, fpw, w2in, fpb, cpb)


@functools.partial(
    pl.kernel,
    out_type=jax.ShapeDtypeStruct((ROWS, D), jnp.float32),
    mesh=plsc.VectorSubcoreMesh(core_axis_name="c", subcore_axis_name="s"),
    scratch_types=[
        pltpu.VMEM((N_CH, CHUNK), jnp.int32),
        pltpu.VMEM((N_CH, CHUNK), jnp.int32),
        pltpu.VMEM((CHUNK,), jnp.float32),
        pltpu.VMEM((CHUNK,), jnp.float32),
        pltpu.VMEM((CHUNK, D), jnp.float32),
        pltpu.VMEM((CHUNK, D), jnp.float32),
        pltpu.VMEM((CHUNK, D), jnp.float32),
        pltpu.VMEM((CHUNK, D), jnp.float32),
        pltpu.SemaphoreType.DMA,
        pltpu.SemaphoreType.DMA,
        pltpu.SemaphoreType.DMA,
        pltpu.SemaphoreType.DMA,
        pltpu.SemaphoreType.DMA,
        pltpu.SemaphoreType.DMA,
        pltpu.SemaphoreType.DMA,
        pltpu.SemaphoreType.DMA,
        pltpu.SemaphoreType.DMA,
    ],
)
def _sc_gather_add(tok_hbm, wf_hbm, notes_hbm, p1_hbm, p2_hbm, out_hbm,
                   idx_t_all, idx_w_all, notes0, notes1,
                   rows1_0, rows1_1, rows2_0, rows2_1,
                   isem, nsem0, nsem1, g1s0, g1s1, g2s0, g2s1, ws0, ws1):
    notes_v = (notes0, notes1)
    rows1 = (rows1_0, rows1_1)
    rows2 = (rows2_0, rows2_1)
    nsem = (nsem0, nsem1)
    g1s = (g1s0, g1s1)
    g2s = (g2s0, g2s1)
    ws = (ws0, ws1)

    wid = lax.axis_index("s") * NC + lax.axis_index("c")
    base_w = wid * PER_W
    crow = wid * N_CH
    lanes = lax.iota(jnp.int32, LANES)

    ct = pltpu.async_copy(tok_hbm.at[pl.ds(crow, N_CH), :], idx_t_all, isem)
    cw = pltpu.async_copy(wf_hbm.at[pl.ds(crow, N_CH), :], idx_w_all, isem)
    ct.wait()
    cw.wait()

    def fire(c, b):
        base = base_w + c * CHUNK
        pltpu.async_copy(notes_hbm.at[pl.ds(base, CHUNK)], notes_v[b], nsem[b])
        pltpu.async_copy(p1_hbm.at[idx_t_all.at[c]], rows1[b], g1s[b])
        pltpu.async_copy(p2_hbm.at[idx_w_all.at[c]], rows2[b], g2s[b])

    def wait_gathers(b):
        pltpu.make_async_copy(notes_hbm.at[pl.ds(0, CHUNK)], notes_v[b], nsem[b]).wait()
        pltpu.make_async_copy(p1_hbm.at[idx_t_all.at[0]], rows1[b], g1s[b]).wait()
        pltpu.make_async_copy(p2_hbm.at[idx_w_all.at[0]], rows2[b], g2s[b]).wait()

    def fire_out(c, b):
        base = base_w + c * CHUNK
        pltpu.async_copy(rows1[b], out_hbm.at[pl.ds(base, CHUNK), :], ws[b])

    def wait_out(b):
        pltpu.make_async_copy(rows1[b], out_hbm.at[pl.ds(0, CHUNK), :], ws[b]).wait()

    def compute(b):
        r1, r2, nv_ref = rows1[b], rows2[b], notes_v[b]

        @plsc.parallel_loop(0, CHUNK, unroll=4)
        def _(row):
            for g in range(D // LANES):
                sl = pl.ds(g * LANES, LANES)
                plsc.addupdate(r1.at[row, sl], r2[row, sl])

        @plsc.parallel_loop(0, CHUNK // LANES, unroll=2)
        def _(q):
            nv = nv_ref[pl.ds(q * LANES, LANES)]
            sl = pl.ds(D - LANES, LANES)
            for r in range(LANES):
                vec = jnp.where(lanes == LANES - 1,
                                jnp.broadcast_to(nv[r], (LANES,)), 0.0)
                plsc.addupdate(r1.at[q * LANES + r, sl], vec)

    fire(0, 0)

    def body(i, carry):
        for bslot in range(2):
            c = 2 * i + bslot
            q = 1 - bslot

            @pl.when(c >= 1)
            def _():
                wait_out(q)

            @pl.when(c + 1 < N_CH)
            def _():
                fire(c + 1, q)

            wait_gathers(bslot)
            compute(bslot)
            fire_out(c, bslot)
        return carry

    lax.fori_loop(0, N_CH // 2, body, 0)
    wait_out(1)


def kernel(token, weighted_factor, weighted_notes, src_table, wf_table,
           fp_W, fp_b, cp_W, cp_b):
    w1 = jnp.pad(cp_W[:D, :], ((0, 0), (0, 1)))
    w2in = jnp.pad(cp_W[D:, :], ((0, 7), (0, 1)))
    fpw = jnp.pad(fp_W, ((0, 0), (0, 7)))
    fpb = jnp.pad(fp_b, (0, 7)).reshape(1, 32)
    cpb = jnp.pad(cp_b, (0, 1)).reshape(1, D)

    p1, p2 = _project_tables(src_table, wf_table, w1, fpw, w2in, fpb, cpb)

    tok = token.reshape(NW * N_CH, CHUNK).astype(jnp.int32)
    wf = weighted_factor.reshape(NW * N_CH, CHUNK).astype(jnp.int32)
    notes = weighted_notes.reshape(ROWS)
    out = _sc_gather_add(tok, wf, notes, p1, p2)
    return out.reshape(B, L, D)
